# Initial kernel scaffold; baseline (speedup 1.0000x reference)
import jax
import jax.numpy as jnp
from jax.experimental import pallas as pl


def kernel(x, edge_node, edge_he, idx_mask, num_hyperedges):
    raise NotImplementedError("write your pallas kernel here")


# speedup vs baseline: 6.5312x; 6.5312x over previous
import functools

import jax
import jax.numpy as jnp
from jax import lax
from jax.experimental import pallas as pl
from jax.experimental.pallas import tpu as pltpu
from jax.experimental.pallas import tpu_sc as plsc

N_NODES = 10000
D = 128
N_EDGES = 320000
N_HE = 5000
N_MASK = 2000

NC = 2
NS = 16
L = 16
NW = NC * NS

ROWS_PER_TILE = 320
HE_PAD = 6144
HE_SLICE = HE_PAD // NS
EDGES_PER_TILE = N_EDGES // NS
MASK_CHUNKS = N_MASK // L


def _token_body(x_ref, o_ref, acc_ref):
    i = pl.program_id(0)

    @pl.when(i == 0)
    def _():
        acc_ref[...] = jnp.zeros_like(acc_ref)

    acc_ref[...] += jnp.sum(x_ref[...], axis=0, keepdims=True)

    @pl.when(i == pl.num_programs(0) - 1)
    def _():
        o_ref[...] = acc_ref[...] * (1.0 / N_NODES)


def _compute_token(x):
    return pl.pallas_call(
        _token_body,
        grid=(10,),
        in_specs=[pl.BlockSpec((N_NODES // 10, D), lambda i: (i, 0))],
        out_specs=pl.BlockSpec((1, D), lambda i: (0, 0)),
        out_shape=jax.ShapeDtypeStruct((1, D), jnp.float32),
        scratch_shapes=[pltpu.VMEM((1, D), jnp.float32)],
    )(x)


_sc_mesh = plsc.VectorSubcoreMesh(
    core_axis_name="c", subcore_axis_name="s", num_cores=NC, num_subcores=NS)


@functools.partial(
    pl.kernel,
    out_type=(
        jax.ShapeDtypeStruct((N_NODES, D), jnp.float32),
        jax.ShapeDtypeStruct((HE_PAD,), jnp.int32),
    ),
    mesh=_sc_mesh,
    scratch_types=[
        pltpu.VMEM((EDGES_PER_TILE,), jnp.int32),
        pltpu.VMEM((HE_PAD,), jnp.int32),
        pltpu.VMEM((NS, HE_SLICE), jnp.int32),
        pltpu.VMEM((ROWS_PER_TILE, D), jnp.float32),
        pltpu.VMEM((D,), jnp.float32),
        pltpu.VMEM((N_MASK,), jnp.int32),
        pltpu.VMEM((HE_SLICE,), jnp.int32),
        pltpu.VMEM((ROWS_PER_TILE,), jnp.int32),
        pltpu.VMEM_SHARED((NS, HE_PAD), jnp.int32),
    ],
    compiler_params=pltpu.CompilerParams(needs_layout_passes=False),
)
def _sc_kernel(x_hbm, ehe_hbm, idxm_hbm, tok_hbm, xm_hbm, he_hbm,
               ehe_v, ind_v, red_v, xblk_v, tok_v, idxm_v, heo_v, fl_v,
               shared):
    c = lax.axis_index("c")
    s = lax.axis_index("s")
    w = c * NS + s

    ones = jnp.ones((L,), jnp.int32)
    zeros = jnp.zeros((L,), jnp.int32)
    iota = lax.iota(jnp.int32, L)

    @pl.when(c == 0)
    def _():
        pltpu.sync_copy(
            ehe_hbm.at[pl.ds(s * EDGES_PER_TILE, EDGES_PER_TILE)], ehe_v)

        def zero_body(i, carry):
            ind_v[pl.ds(i * L, L)] = zeros
            return carry
        lax.fori_loop(0, HE_PAD // L, zero_body, 0)

        def scat_body(i, carry):
            idx = ehe_v[pl.ds(i * L, L)]
            plsc.store_scatter(ind_v, [idx], ones)
            return carry
        lax.fori_loop(0, EDGES_PER_TILE // L, scat_body, 0)

        pltpu.sync_copy(ind_v, shared.at[s])

    plsc.subcore_barrier()

    @pl.when(c == 0)
    def _():
        pltpu.sync_copy(shared.at[:, pl.ds(s * HE_SLICE, HE_SLICE)], red_v)

        def red_body(k, carry):
            acc = red_v[0, pl.ds(k * L, L)]
            for j in range(1, NS):
                acc = acc | red_v[j, pl.ds(k * L, L)]
            heo_v[pl.ds(k * L, L)] = acc
            return carry
        lax.fori_loop(0, HE_SLICE // L, red_body, 0)

        pltpu.sync_copy(heo_v, he_hbm.at[pl.ds(s * HE_SLICE, HE_SLICE)])

    pltpu.sync_copy(tok_hbm, tok_v)
    pltpu.sync_copy(idxm_hbm, idxm_v)
    r0 = jnp.minimum(w * ROWS_PER_TILE, N_NODES - ROWS_PER_TILE)
    pltpu.sync_copy(x_hbm.at[pl.ds(r0, ROWS_PER_TILE)], xblk_v)

    for i in range(ROWS_PER_TILE // L):
        fl_v[pl.ds(i * L, L)] = zeros

    def mask_body(i, carry):
        iv = idxm_v[pl.ds(i * L, L)]
        rel = iv - r0
        inb = (rel >= 0) & (rel < ROWS_PER_TILE)
        relc = jnp.clip(rel, 0, ROWS_PER_TILE - 1)
        plsc.store_scatter(fl_v, [relc], ones, mask=inb)
        return carry
    lax.fori_loop(0, MASK_CHUNKS, mask_body, 0)

    def sel_body(r, carry):
        fl = plsc.load_gather(fl_v, [jnp.full((L,), 0, jnp.int32) + r])
        pred = fl > 0
        for ch in range(D // L):
            xv = xblk_v[r, pl.ds(ch * L, L)]
            xblk_v[r, pl.ds(ch * L, L)] = jnp.where(
                pred, tok_v[pl.ds(ch * L, L)], xv)
        return carry
    lax.fori_loop(0, ROWS_PER_TILE, sel_body, 0)

    pltpu.sync_copy(xblk_v, xm_hbm.at[pl.ds(r0, ROWS_PER_TILE)])


def kernel(x, edge_node, edge_he, idx_mask, num_hyperedges):
    token = _compute_token(x).reshape(D)
    xm, he_pad = _sc_kernel(x, edge_he, idx_mask, token)
    return xm, he_pad[:N_HE]


# speedup vs baseline: 6.9177x; 1.0592x over previous
import functools

import jax
import jax.numpy as jnp
from jax import lax
from jax.experimental import pallas as pl
from jax.experimental.pallas import tpu as pltpu
from jax.experimental.pallas import tpu_sc as plsc

N_NODES = 10000
D = 128
N_EDGES = 320000
N_HE = 5000
N_MASK = 2000

NC = 2
NS = 16
L = 16
NW = NC * NS

ROWS_PER_TILE = 320
HE_PAD = 6144
HE_SLICE = HE_PAD // NS
EDGES_PER_TILE = N_EDGES // NS
MASK_CHUNKS = N_MASK // L


def _token_body(x_ref, o_ref, acc_ref):
    i = pl.program_id(0)

    @pl.when(i == 0)
    def _():
        acc_ref[...] = jnp.zeros_like(acc_ref)

    acc_ref[...] += jnp.sum(x_ref[...], axis=0, keepdims=True)

    @pl.when(i == pl.num_programs(0) - 1)
    def _():
        o_ref[...] = acc_ref[...] * (1.0 / N_NODES)


def _compute_token(x):
    return pl.pallas_call(
        _token_body,
        grid=(10,),
        in_specs=[pl.BlockSpec((N_NODES // 10, D), lambda i: (i, 0))],
        out_specs=pl.BlockSpec((1, D), lambda i: (0, 0)),
        out_shape=jax.ShapeDtypeStruct((1, D), jnp.float32),
        scratch_shapes=[pltpu.VMEM((1, D), jnp.float32)],
    )(x)


_sc_mesh = plsc.VectorSubcoreMesh(
    core_axis_name="c", subcore_axis_name="s", num_cores=NC, num_subcores=NS)


@functools.partial(
    pl.kernel,
    out_type=(
        jax.ShapeDtypeStruct((N_NODES, D), jnp.float32),
        jax.ShapeDtypeStruct((HE_PAD,), jnp.int32),
    ),
    mesh=_sc_mesh,
    scratch_types=[
        pltpu.VMEM((EDGES_PER_TILE,), jnp.int32),
        pltpu.VMEM((HE_PAD,), jnp.int32),
        pltpu.VMEM((NS, HE_SLICE), jnp.int32),
        pltpu.VMEM((ROWS_PER_TILE, D), jnp.float32),
        pltpu.VMEM((D,), jnp.float32),
        pltpu.VMEM((N_MASK,), jnp.int32),
        pltpu.VMEM((HE_SLICE,), jnp.int32),
        pltpu.VMEM((ROWS_PER_TILE,), jnp.int32),
        pltpu.VMEM_SHARED((NS, HE_PAD), jnp.int32),
    ],
    compiler_params=pltpu.CompilerParams(needs_layout_passes=False),
)
def _sc_kernel(x_hbm, ehe_hbm, idxm_hbm, tok_hbm, xm_hbm, he_hbm,
               ehe_v, ind_v, red_v, xblk_v, tok_v, idxm_v, heo_v, fl_v,
               shared):
    c = lax.axis_index("c")
    s = lax.axis_index("s")
    w = c * NS + s

    ones = jnp.ones((L,), jnp.int32)
    zeros = jnp.zeros((L,), jnp.int32)
    iota = lax.iota(jnp.int32, L)

    @pl.when(c == 0)
    def _():
        pltpu.sync_copy(
            ehe_hbm.at[pl.ds(s * EDGES_PER_TILE, EDGES_PER_TILE)], ehe_v)

        def zero_body(i, carry):
            ind_v[pl.ds(i * L, L)] = zeros
            return carry
        lax.fori_loop(0, HE_PAD // L, zero_body, 0)

        def scat_body(i, carry):
            for k in range(4):
                idx = ehe_v[pl.ds(i * (4 * L) + k * L, L)]
                plsc.store_scatter(ind_v, [idx], ones)
            return carry
        lax.fori_loop(0, EDGES_PER_TILE // (4 * L), scat_body, 0)

        pltpu.sync_copy(ind_v, shared.at[s])

    plsc.subcore_barrier()

    @pl.when(c == 0)
    def _():
        pltpu.sync_copy(shared.at[:, pl.ds(s * HE_SLICE, HE_SLICE)], red_v)

        def red_body(k, carry):
            acc = red_v[0, pl.ds(k * L, L)]
            for j in range(1, NS):
                acc = acc | red_v[j, pl.ds(k * L, L)]
            heo_v[pl.ds(k * L, L)] = acc
            return carry
        lax.fori_loop(0, HE_SLICE // L, red_body, 0)

        pltpu.sync_copy(heo_v, he_hbm.at[pl.ds(s * HE_SLICE, HE_SLICE)])

    pltpu.sync_copy(tok_hbm, tok_v)
    pltpu.sync_copy(idxm_hbm, idxm_v)
    r0 = jnp.minimum(w * ROWS_PER_TILE, N_NODES - ROWS_PER_TILE)
    pltpu.sync_copy(x_hbm.at[pl.ds(r0, ROWS_PER_TILE)], xblk_v)

    for i in range(ROWS_PER_TILE // L):
        fl_v[pl.ds(i * L, L)] = zeros

    def mask_body(i, carry):
        iv = idxm_v[pl.ds(i * L, L)]
        rel = iv - r0
        inb = (rel >= 0) & (rel < ROWS_PER_TILE)
        relc = jnp.clip(rel, 0, ROWS_PER_TILE - 1)
        plsc.store_scatter(fl_v, [relc], ones, mask=inb)
        return carry
    lax.fori_loop(0, MASK_CHUNKS, mask_body, 0)

    tvs = [tok_v[pl.ds(ch * L, L)] for ch in range(D // L)]

    def sel_body(r, carry):
        fl = plsc.load_gather(fl_v, [jnp.full((L,), 0, jnp.int32) + r])
        cnt = jnp.sum(fl)

        @pl.when(cnt > 0)
        def _():
            for ch in range(D // L):
                xblk_v[r, pl.ds(ch * L, L)] = tvs[ch]
        return carry
    lax.fori_loop(0, ROWS_PER_TILE, sel_body, 0)

    pltpu.sync_copy(xblk_v, xm_hbm.at[pl.ds(r0, ROWS_PER_TILE)])


def kernel(x, edge_node, edge_he, idx_mask, num_hyperedges):
    token = _compute_token(x).reshape(D)
    xm, he_pad = _sc_kernel(x, edge_he, idx_mask, token)
    return xm, he_pad[:N_HE]


# speedup vs baseline: 8.4640x; 1.2235x over previous
import functools

import jax
import jax.numpy as jnp
from jax import lax
from jax.experimental import pallas as pl
from jax.experimental.pallas import tpu as pltpu
from jax.experimental.pallas import tpu_sc as plsc

N_NODES = 10000
D = 128
N_EDGES = 320000
N_HE = 5000
N_MASK = 2000

NC = 2
NS = 16
L = 16

ROWS_PER_TILE = 640
BLK = 128
NBLK = ROWS_PER_TILE // BLK
FL_PAD = 640
HE_PAD = 6144
HE_SLICE = HE_PAD // NS
EDGES_PER_TILE = N_EDGES // NS
MASK_CHUNKS = N_MASK // L


def _token_body(x_ref, o_ref, acc_ref):
    i = pl.program_id(0)

    @pl.when(i == 0)
    def _():
        acc_ref[...] = jnp.zeros_like(acc_ref)

    acc_ref[...] += jnp.sum(x_ref[...], axis=0, keepdims=True)

    @pl.when(i == pl.num_programs(0) - 1)
    def _():
        o_ref[...] = acc_ref[...] * (1.0 / N_NODES)


def _compute_token(x):
    return pl.pallas_call(
        _token_body,
        grid=(5,),
        in_specs=[pl.BlockSpec((N_NODES // 5, D), lambda i: (i, 0))],
        out_specs=pl.BlockSpec((1, D), lambda i: (0, 0)),
        out_shape=jax.ShapeDtypeStruct((1, D), jnp.float32),
        scratch_shapes=[pltpu.VMEM((1, D), jnp.float32)],
    )(x)


_sc_mesh = plsc.VectorSubcoreMesh(
    core_axis_name="c", subcore_axis_name="s", num_cores=NC, num_subcores=NS)


@functools.partial(
    pl.kernel,
    out_type=(
        jax.ShapeDtypeStruct((N_NODES, D), jnp.float32),
        jax.ShapeDtypeStruct((N_HE,), jnp.int32),
    ),
    mesh=_sc_mesh,
    scratch_types=[
        pltpu.VMEM((EDGES_PER_TILE,), jnp.int32),
        pltpu.VMEM((HE_PAD,), jnp.int32),
        pltpu.VMEM((NS, HE_SLICE), jnp.int32),
        pltpu.VMEM((BLK, D), jnp.float32),
        pltpu.VMEM((BLK, D), jnp.float32),
        pltpu.VMEM((BLK, D), jnp.float32),
        pltpu.VMEM((D,), jnp.float32),
        pltpu.VMEM((N_MASK,), jnp.int32),
        pltpu.VMEM((HE_SLICE,), jnp.int32),
        pltpu.VMEM((FL_PAD,), jnp.int32),
        pltpu.VMEM_SHARED((NS, HE_PAD), jnp.int32),
        pltpu.SemaphoreType.DMA,
        pltpu.SemaphoreType.DMA,
        pltpu.SemaphoreType.DMA,
        pltpu.SemaphoreType.DMA,
        pltpu.SemaphoreType.DMA,
        pltpu.SemaphoreType.DMA,
    ],
    compiler_params=pltpu.CompilerParams(needs_layout_passes=False),
)
def _sc_kernel(x_hbm, ehe_hbm, idxm_hbm, tok_hbm, xm_hbm, he_hbm,
               ehe_v, ind_v, red_v, xb0, xb1, xb2, tok_v, idxm_v, heo_v,
               fl_v, shared, isem0, isem1, isem2, osem0, osem1, osem2):
    c = lax.axis_index("c")
    s = lax.axis_index("s")

    ones = jnp.ones((L,), jnp.int32)
    zeros = jnp.zeros((L,), jnp.int32)

    @pl.when(c == 0)
    def _():
        pltpu.sync_copy(
            ehe_hbm.at[pl.ds(s * EDGES_PER_TILE, EDGES_PER_TILE)], ehe_v)

        def zero_body(i, carry):
            ind_v[pl.ds(i * L, L)] = zeros
            return carry
        lax.fori_loop(0, HE_PAD // L, zero_body, 0)

        def scat_body(i, carry):
            for k in range(4):
                idx = ehe_v[pl.ds(i * (4 * L) + k * L, L)]
                plsc.store_scatter(ind_v, [idx], ones)
            return carry
        lax.fori_loop(0, EDGES_PER_TILE // (4 * L), scat_body, 0)

        pltpu.sync_copy(ind_v, shared.at[s])

    plsc.subcore_barrier()

    @pl.when(c == 0)
    def _():
        pltpu.sync_copy(shared.at[:, pl.ds(s * HE_SLICE, HE_SLICE)], red_v)

        def red_body(k, carry):
            acc = red_v[0, pl.ds(k * L, L)]
            for j in range(1, NS):
                acc = acc | red_v[j, pl.ds(k * L, L)]
            heo_v[pl.ds(k * L, L)] = acc
            return carry
        lax.fori_loop(0, HE_SLICE // L, red_body, 0)

        @pl.when(s <= 12)
        def _():
            pltpu.sync_copy(heo_v, he_hbm.at[pl.ds(s * HE_SLICE, HE_SLICE)])

        @pl.when(s == 13)
        def _():
            pltpu.sync_copy(heo_v.at[pl.ds(0, 8)],
                            he_hbm.at[pl.ds(13 * HE_SLICE, 8)])

    @pl.when(c == 1)
    def _():
        pltpu.sync_copy(tok_hbm, tok_v)
        pltpu.sync_copy(idxm_hbm, idxm_v)
        r0 = jnp.minimum(s * ROWS_PER_TILE, N_NODES - ROWS_PER_TILE)

        for i in range(FL_PAD // L):
            fl_v[pl.ds(i * L, L)] = zeros

        def mask_body(i, carry):
            iv = idxm_v[pl.ds(i * L, L)]
            rel = iv - r0
            inb = (rel >= 0) & (rel < ROWS_PER_TILE)
            relc = jnp.clip(rel, 0, ROWS_PER_TILE - 1)
            plsc.store_scatter(fl_v, [relc], ones, mask=inb)
            return carry
        lax.fori_loop(0, MASK_CHUNKS, mask_body, 0)

        tvs = [tok_v[pl.ds(ch * L, L)] for ch in range(D // L)]
        bufs = [xb0, xb1, xb2]
        isems = [isem0, isem1, isem2]
        osems = [osem0, osem1, osem2]
        NBUF = 3

        def start_in(g):
            return pltpu.async_copy(
                x_hbm.at[pl.ds(r0 + g * BLK, BLK)], bufs[g % NBUF],
                isems[g % NBUF])

        def process(g):
            buf = bufs[g % NBUF]

            def sel_body(r, carry):
                fl = plsc.load_gather(
                    fl_v, [jnp.full((L,), g * BLK, jnp.int32) + r])
                cnt = jnp.sum(fl)

                @pl.when(cnt > 0)
                def _():
                    for ch in range(D // L):
                        buf[r, pl.ds(ch * L, L)] = tvs[ch]
                return carry
            lax.fori_loop(0, BLK, sel_body, 0)

        def start_out(g):
            return pltpu.async_copy(
                bufs[g % NBUF], xm_hbm.at[pl.ds(r0 + g * BLK, BLK)],
                osems[g % NBUF])

        in_cp = {0: start_in(0), 1: start_in(1)}
        out_cp = {}
        for g in range(NBLK):
            in_cp[g].wait()
            process(g)
            out_cp[g] = start_out(g)
            if g + 2 < NBLK:
                if g - 1 >= 0:
                    out_cp[g - 1].wait()
                in_cp[g + 2] = start_in(g + 2)
        for g in range(max(0, NBLK - 3), NBLK):
            out_cp[g].wait()


def kernel(x, edge_node, edge_he, idx_mask, num_hyperedges):
    token = _compute_token(x).reshape(D)
    xm, he = _sc_kernel(x, edge_he, idx_mask, token)
    return xm, he


# speedup vs baseline: 10.0111x; 1.1828x over previous
import functools

import jax
import jax.numpy as jnp
from jax import lax
from jax.experimental import pallas as pl
from jax.experimental.pallas import tpu as pltpu
from jax.experimental.pallas import tpu_sc as plsc

N_NODES = 10000
D = 128
N_EDGES = 320000
N_HE = 5000
N_MASK = 2000

NC = 2
NS = 16
L = 16

ROWS_PER_TILE = 640
BLK = 128
NBLK = ROWS_PER_TILE // BLK
FL_PAD = 640
HE_PAD = 6144
HE_SLICE = HE_PAD // NS
EDGES_PER_TILE = N_EDGES // NS
MASK_CHUNKS = N_MASK // L


def _token_body(x_ref, o_ref, acc_ref):
    i = pl.program_id(0)

    @pl.when(i == 0)
    def _():
        acc_ref[...] = jnp.zeros_like(acc_ref)

    acc_ref[...] += jnp.sum(x_ref[...], axis=0, keepdims=True)

    @pl.when(i == pl.num_programs(0) - 1)
    def _():
        o_ref[...] = acc_ref[...] * (1.0 / N_NODES)


def _compute_token(x):
    return pl.pallas_call(
        _token_body,
        grid=(5,),
        in_specs=[pl.BlockSpec((N_NODES // 5, D), lambda i: (i, 0))],
        out_specs=pl.BlockSpec((1, D), lambda i: (0, 0)),
        out_shape=jax.ShapeDtypeStruct((1, D), jnp.float32),
        scratch_shapes=[pltpu.VMEM((1, D), jnp.float32)],
    )(x)


_sc_mesh = plsc.VectorSubcoreMesh(
    core_axis_name="c", subcore_axis_name="s", num_cores=NC, num_subcores=NS)


@functools.partial(
    pl.kernel,
    out_type=(
        jax.ShapeDtypeStruct((N_NODES, D), jnp.float32),
        jax.ShapeDtypeStruct((N_HE,), jnp.int32),
    ),
    mesh=_sc_mesh,
    scratch_types=[
        pltpu.VMEM((EDGES_PER_TILE,), jnp.int32),
        pltpu.VMEM((HE_PAD,), jnp.int32),
        pltpu.VMEM((NS, HE_SLICE), jnp.int32),
        pltpu.VMEM((BLK, D), jnp.float32),
        pltpu.VMEM((BLK, D), jnp.float32),
        pltpu.VMEM((BLK, D), jnp.float32),
        pltpu.VMEM((D,), jnp.float32),
        pltpu.VMEM((N_MASK,), jnp.int32),
        pltpu.VMEM((HE_SLICE,), jnp.int32),
        pltpu.VMEM((FL_PAD,), jnp.int32),
        pltpu.VMEM_SHARED((NS, HE_PAD), jnp.int32),
        pltpu.SemaphoreType.DMA,
        pltpu.SemaphoreType.DMA,
        pltpu.SemaphoreType.DMA,
        pltpu.SemaphoreType.DMA,
        pltpu.SemaphoreType.DMA,
        pltpu.SemaphoreType.DMA,
    ],
    compiler_params=pltpu.CompilerParams(needs_layout_passes=False),
)
def _sc_kernel(x_hbm, ehe_hbm, idxm_hbm, tok_hbm, xm_hbm, he_hbm,
               ehe_v, ind_v, red_v, xb0, xb1, xb2, tok_v, idxm_v, heo_v,
               fl_v, shared, isem0, isem1, isem2, osem0, osem1, osem2):
    c = lax.axis_index("c")
    s = lax.axis_index("s")

    ones = jnp.ones((L,), jnp.int32)
    zeros = jnp.zeros((L,), jnp.int32)

    @pl.when(c == 0)
    def _():
        pltpu.sync_copy(
            ehe_hbm.at[pl.ds(s * EDGES_PER_TILE, EDGES_PER_TILE)], ehe_v)

        def zero_body(i, carry):
            ind_v[pl.ds(i * L, L)] = zeros
            return carry
        lax.fori_loop(0, HE_PAD // L, zero_body, 0)

        def scat_body(i, carry):
            for k in range(4):
                idx = ehe_v[pl.ds(i * (4 * L) + k * L, L)]
                plsc.store_scatter(ind_v, [idx], ones)
            return carry
        lax.fori_loop(0, EDGES_PER_TILE // (4 * L), scat_body, 0)

        pltpu.sync_copy(ind_v, shared.at[s])

    plsc.subcore_barrier()

    @pl.when(c == 0)
    def _():
        pltpu.sync_copy(shared.at[:, pl.ds(s * HE_SLICE, HE_SLICE)], red_v)

        def red_body(k, carry):
            acc = red_v[0, pl.ds(k * L, L)]
            for j in range(1, NS):
                acc = acc | red_v[j, pl.ds(k * L, L)]
            heo_v[pl.ds(k * L, L)] = acc
            return carry
        lax.fori_loop(0, HE_SLICE // L, red_body, 0)

        @pl.when(s <= 12)
        def _():
            pltpu.sync_copy(heo_v, he_hbm.at[pl.ds(s * HE_SLICE, HE_SLICE)])

        @pl.when(s == 13)
        def _():
            pltpu.sync_copy(heo_v.at[pl.ds(0, 8)],
                            he_hbm.at[pl.ds(13 * HE_SLICE, 8)])

    @pl.when(c == 1)
    def _():
        pltpu.sync_copy(tok_hbm, tok_v)
        pltpu.sync_copy(idxm_hbm, idxm_v)
        r0 = jnp.minimum(s * ROWS_PER_TILE, N_NODES - ROWS_PER_TILE)

        for i in range(FL_PAD // L):
            fl_v[pl.ds(i * L, L)] = zeros

        def mask_body(i, carry):
            iv = idxm_v[pl.ds(i * L, L)]
            rel = iv - r0
            inb = (rel >= 0) & (rel < ROWS_PER_TILE)
            relc = jnp.clip(rel, 0, ROWS_PER_TILE - 1)
            plsc.store_scatter(fl_v, [relc], ones, mask=inb)
            return carry
        lax.fori_loop(0, MASK_CHUNKS, mask_body, 0)

        tvs = [tok_v[pl.ds(ch * L, L)] for ch in range(D // L)]
        bufs = [xb0, xb1, xb2]
        isems = [isem0, isem1, isem2]
        osems = [osem0, osem1, osem2]
        NBUF = 3

        def start_in(g):
            return pltpu.async_copy(
                x_hbm.at[pl.ds(r0 + g * BLK, BLK)], bufs[g % NBUF],
                isems[g % NBUF])

        def process(g):
            buf = bufs[g % NBUF]

            def sel_body(r, carry):
                for u in range(2):
                    fl = plsc.load_gather(
                        fl_v,
                        [jnp.full((L,), g * BLK + u, jnp.int32) + 2 * r])
                    pred = fl > 0
                    for ch in range(D // L):
                        plsc.store_compressed(
                            buf.at[2 * r + u, pl.ds(ch * L, L)],
                            tvs[ch], mask=pred)
                return carry
            lax.fori_loop(0, BLK // 2, sel_body, 0)

        def start_out(g):
            return pltpu.async_copy(
                bufs[g % NBUF], xm_hbm.at[pl.ds(r0 + g * BLK, BLK)],
                osems[g % NBUF])

        in_cp = {0: start_in(0), 1: start_in(1)}
        out_cp = {}
        for g in range(NBLK):
            in_cp[g].wait()
            process(g)
            out_cp[g] = start_out(g)
            if g + 2 < NBLK:
                if g - 1 >= 0:
                    out_cp[g - 1].wait()
                in_cp[g + 2] = start_in(g + 2)
        for g in range(max(0, NBLK - 3), NBLK):
            out_cp[g].wait()


def kernel(x, edge_node, edge_he, idx_mask, num_hyperedges):
    token = _compute_token(x).reshape(D)
    xm, he = _sc_kernel(x, edge_he, idx_mask, token)
    return xm, he


# speedup vs baseline: 10.9215x; 1.0909x over previous
import functools

import jax
import jax.numpy as jnp
from jax import lax
from jax.experimental import pallas as pl
from jax.experimental.pallas import tpu as pltpu
from jax.experimental.pallas import tpu_sc as plsc

N_NODES = 10000
D = 128
N_EDGES = 320000
N_HE = 5000
N_MASK = 2000

NC = 2
NS = 16
L = 16

ROWS_PER_TILE = 640
BLK = 128
NBLK = ROWS_PER_TILE // BLK
FL_PAD = 640
HE_PAD = 6144
HE_SLICE = HE_PAD // NS
EDGES_PER_TILE = N_EDGES // NS
MASK_CHUNKS = N_MASK // L


def _token_body(x_ref, o_ref, acc_ref):
    i = pl.program_id(0)

    @pl.when(i == 0)
    def _():
        acc_ref[...] = jnp.zeros_like(acc_ref)

    acc_ref[...] += jnp.sum(x_ref[...], axis=0, keepdims=True)

    @pl.when(i == pl.num_programs(0) - 1)
    def _():
        o_ref[...] = acc_ref[...] * (1.0 / N_NODES)


def _compute_token(x):
    return pl.pallas_call(
        _token_body,
        grid=(2,),
        in_specs=[pl.BlockSpec((N_NODES // 2, D), lambda i: (i, 0))],
        out_specs=pl.BlockSpec((1, D), lambda i: (0, 0)),
        out_shape=jax.ShapeDtypeStruct((1, D), jnp.float32),
        scratch_shapes=[pltpu.VMEM((1, D), jnp.float32)],
    )(x)


_sc_mesh = plsc.VectorSubcoreMesh(
    core_axis_name="c", subcore_axis_name="s", num_cores=NC, num_subcores=NS)


@functools.partial(
    pl.kernel,
    out_type=(
        jax.ShapeDtypeStruct((N_NODES, D), jnp.float32),
        jax.ShapeDtypeStruct((N_HE,), jnp.int32),
    ),
    mesh=_sc_mesh,
    scratch_types=[
        pltpu.VMEM((EDGES_PER_TILE,), jnp.int32),
        pltpu.VMEM((HE_PAD,), jnp.int32),
        pltpu.VMEM((NS, HE_SLICE), jnp.int32),
        pltpu.VMEM((BLK, D), jnp.float32),
        pltpu.VMEM((BLK, D), jnp.float32),
        pltpu.VMEM((BLK, D), jnp.float32),
        pltpu.VMEM((D,), jnp.float32),
        pltpu.VMEM((N_MASK,), jnp.int32),
        pltpu.VMEM((HE_SLICE,), jnp.int32),
        pltpu.VMEM((FL_PAD,), jnp.int32),
        pltpu.VMEM_SHARED((NS, HE_PAD), jnp.int32),
        pltpu.SemaphoreType.DMA,
        pltpu.SemaphoreType.DMA,
        pltpu.SemaphoreType.DMA,
        pltpu.SemaphoreType.DMA,
        pltpu.SemaphoreType.DMA,
        pltpu.SemaphoreType.DMA,
    ],
    compiler_params=pltpu.CompilerParams(needs_layout_passes=False),
)
def _sc_kernel(x_hbm, ehe_hbm, idxm_hbm, tok_hbm, xm_hbm, he_hbm,
               ehe_v, ind_v, red_v, xb0, xb1, xb2, tok_v, idxm_v, heo_v,
               fl_v, shared, isem0, isem1, isem2, osem0, osem1, osem2):
    c = lax.axis_index("c")
    s = lax.axis_index("s")

    ones = jnp.ones((L,), jnp.int32)
    zeros = jnp.zeros((L,), jnp.int32)

    @pl.when(c == 0)
    def _():
        pltpu.sync_copy(
            ehe_hbm.at[pl.ds(s * EDGES_PER_TILE, EDGES_PER_TILE)], ehe_v)

        def zero_body(i, carry):
            ind_v[pl.ds(i * L, L)] = zeros
            return carry
        lax.fori_loop(0, HE_PAD // L, zero_body, 0)

        def scat_body(i, carry):
            for k in range(10):
                idx = ehe_v[pl.ds(i * (10 * L) + k * L, L)]
                plsc.store_scatter(ind_v, [idx], ones)
            return carry
        lax.fori_loop(0, EDGES_PER_TILE // (10 * L), scat_body, 0)

        pltpu.sync_copy(ind_v, shared.at[s])

    plsc.subcore_barrier()

    @pl.when(c == 0)
    def _():
        pltpu.sync_copy(shared.at[:, pl.ds(s * HE_SLICE, HE_SLICE)], red_v)

        def red_body(k, carry):
            acc = red_v[0, pl.ds(k * L, L)]
            for j in range(1, NS):
                acc = acc | red_v[j, pl.ds(k * L, L)]
            heo_v[pl.ds(k * L, L)] = acc
            return carry
        lax.fori_loop(0, HE_SLICE // L, red_body, 0)

        @pl.when(s <= 12)
        def _():
            pltpu.sync_copy(heo_v, he_hbm.at[pl.ds(s * HE_SLICE, HE_SLICE)])

        @pl.when(s == 13)
        def _():
            pltpu.sync_copy(heo_v.at[pl.ds(0, 8)],
                            he_hbm.at[pl.ds(13 * HE_SLICE, 8)])

    @pl.when(c == 1)
    def _():
        r0 = jnp.minimum(s * ROWS_PER_TILE, N_NODES - ROWS_PER_TILE)

        def start_in_early(g, buf, sem):
            return pltpu.async_copy(
                x_hbm.at[pl.ds(r0 + g * BLK, BLK)], buf, sem)

        in0 = start_in_early(0, xb0, isem0)
        in1 = start_in_early(1, xb1, isem1)
        pltpu.sync_copy(tok_hbm, tok_v)
        pltpu.sync_copy(idxm_hbm, idxm_v)

        for i in range(FL_PAD // L):
            fl_v[pl.ds(i * L, L)] = zeros

        def mask_body(i, carry):
            iv = idxm_v[pl.ds(i * L, L)]
            rel = iv - r0
            inb = (rel >= 0) & (rel < ROWS_PER_TILE)
            relc = jnp.clip(rel, 0, ROWS_PER_TILE - 1)
            plsc.store_scatter(fl_v, [relc], ones, mask=inb)
            return carry
        lax.fori_loop(0, MASK_CHUNKS, mask_body, 0)

        tvs = [tok_v[pl.ds(ch * L, L)] for ch in range(D // L)]
        bufs = [xb0, xb1, xb2]
        isems = [isem0, isem1, isem2]
        osems = [osem0, osem1, osem2]
        NBUF = 3

        def start_in(g):
            return pltpu.async_copy(
                x_hbm.at[pl.ds(r0 + g * BLK, BLK)], bufs[g % NBUF],
                isems[g % NBUF])

        def process(g):
            buf = bufs[g % NBUF]

            def sel_body(r, carry):
                for u in range(2):
                    fl = plsc.load_gather(
                        fl_v,
                        [jnp.full((L,), g * BLK + u, jnp.int32) + 2 * r])
                    pred = fl > 0
                    for ch in range(D // L):
                        plsc.store_compressed(
                            buf.at[2 * r + u, pl.ds(ch * L, L)],
                            tvs[ch], mask=pred)
                return carry
            lax.fori_loop(0, BLK // 2, sel_body, 0)

        def start_out(g):
            return pltpu.async_copy(
                bufs[g % NBUF], xm_hbm.at[pl.ds(r0 + g * BLK, BLK)],
                osems[g % NBUF])

        in_cp = {0: in0, 1: in1}
        out_cp = {}
        for g in range(NBLK):
            in_cp[g].wait()
            process(g)
            out_cp[g] = start_out(g)
            if g + 2 < NBLK:
                if g - 1 >= 0:
                    out_cp[g - 1].wait()
                in_cp[g + 2] = start_in(g + 2)
        for g in range(max(0, NBLK - 3), NBLK):
            out_cp[g].wait()


def kernel(x, edge_node, edge_he, idx_mask, num_hyperedges):
    token = _compute_token(x).reshape(D)
    xm, he = _sc_kernel(x, edge_he, idx_mask, token)
    return xm, he
